# Initial kernel scaffold; baseline (speedup 1.0000x reference)
import jax
import jax.numpy as jnp
from jax.experimental import pallas as pl


def kernel(inputs, P):
    raise NotImplementedError("write your pallas kernel here")


# speedup vs baseline: 3.4244x; 3.4244x over previous
import jax
import jax.numpy as jnp
from jax.experimental import pallas as pl
from jax.experimental.pallas import tpu as pltpu

_SEQ_BLK = 512


def _bcast_copy(p_ref, o_ref):
    o_ref[0] = p_ref[...]


def kernel(inputs, P):
    b, s = inputs.shape
    s_p, d = P.shape
    return pl.pallas_call(
        _bcast_copy,
        grid=(s // _SEQ_BLK, b),
        in_specs=[pl.BlockSpec((_SEQ_BLK, d), lambda i, j: (i, 0))],
        out_specs=pl.BlockSpec((1, _SEQ_BLK, d), lambda i, j: (j, i, 0)),
        out_shape=jax.ShapeDtypeStruct((b, s, d), P.dtype),
        compiler_params=pltpu.CompilerParams(
            dimension_semantics=("arbitrary", "arbitrary"),
        ),
    )(P)


# speedup vs baseline: 4.1698x; 1.2177x over previous
import jax
import jax.numpy as jnp
from jax.experimental import pallas as pl
from jax.experimental.pallas import tpu as pltpu

_SEQ_BLK = 1024


def _bcast_copy(p_ref, o_ref):
    o_ref[0] = p_ref[...]


def kernel(inputs, P):
    b, s = inputs.shape
    s_p, d = P.shape
    return pl.pallas_call(
        _bcast_copy,
        grid=(s // _SEQ_BLK, b),
        in_specs=[pl.BlockSpec((_SEQ_BLK, d), lambda i, j: (i, 0))],
        out_specs=pl.BlockSpec((1, _SEQ_BLK, d), lambda i, j: (j, i, 0)),
        out_shape=jax.ShapeDtypeStruct((b, s, d), P.dtype),
        compiler_params=pltpu.CompilerParams(
            dimension_semantics=("arbitrary", "arbitrary"),
        ),
    )(P)


# speedup vs baseline: 4.5402x; 1.0888x over previous
import jax
import jax.numpy as jnp
from jax.experimental import pallas as pl
from jax.experimental.pallas import tpu as pltpu

_SEQ_BLK = 2048


def _bcast_copy(p_ref, o_ref):
    o_ref[0] = p_ref[...]


def kernel(inputs, P):
    b, s = inputs.shape
    s_p, d = P.shape
    return pl.pallas_call(
        _bcast_copy,
        grid=(s // _SEQ_BLK, b),
        in_specs=[pl.BlockSpec((_SEQ_BLK, d), lambda i, j: (i, 0))],
        out_specs=pl.BlockSpec((1, _SEQ_BLK, d), lambda i, j: (j, i, 0)),
        out_shape=jax.ShapeDtypeStruct((b, s, d), P.dtype),
        compiler_params=pltpu.CompilerParams(
            dimension_semantics=("arbitrary", "arbitrary"),
        ),
    )(P)
